# baseline (speedup 1.0000x reference)
import functools

import jax
import jax.numpy as jnp
from jax import lax
from jax.experimental import pallas as pl
from jax.experimental.pallas import tpu as pltpu
from jax.experimental.pallas import tpu_sc as plsc

B = 16
S = 2048
D = 1024
NC = 2
NS = 16
NW = NC * NS
HALF = S // 2
R = 64
NCHUNK = HALF // R


def _pos_embed_body(x_hbm, pose_hbm, out_hbm, xrow_v, buf_v):
    c = lax.axis_index("c")
    s = lax.axis_index("s")
    wid = c * NS + s
    i = wid // 2
    h = wid % 2
    j0 = h * HALF

    pltpu.sync_copy(x_hbm.at[pl.ds(i * S, S)], xrow_v)

    def scan_body(k, acc):
        v = xrow_v[pl.ds(k * 16, 16)]
        idx = lax.iota(jnp.int32, 16) + k * 16
        cand = jnp.where(v != 0, idx, S)
        return jnp.minimum(acc, cand)

    acc = lax.fori_loop(0, S // 16, scan_body, jnp.full((16,), S, jnp.int32))
    m = jnp.int32(S)
    for l in range(16):
        m = jnp.minimum(m, acc[l])
    fi = jnp.where(m >= S, 0, m)

    start = j0 - fi + S

    def chunk_body(t, _):
        src_off = pl.multiple_of((start + t * R) * D, 8)
        dst_off = pl.multiple_of((i * S + j0 + t * R) * D, 8)
        pltpu.sync_copy(pose_hbm.at[pl.ds(src_off, R * D)], buf_v)
        pltpu.sync_copy(buf_v, out_hbm.at[pl.ds(dst_off, R * D)])
        return 0

    lax.fori_loop(0, NCHUNK, chunk_body, 0)


_pos_embed = functools.partial(
    pl.kernel,
    out_type=jax.ShapeDtypeStruct((B * S * D,), jnp.float32),
    mesh=plsc.VectorSubcoreMesh(core_axis_name="c", subcore_axis_name="s"),
    scratch_types=[
        pltpu.VMEM((S,), jnp.int32),
        pltpu.VMEM((R * D,), jnp.float32),
    ],
)(_pos_embed_body)


@jax.jit
def kernel(x, pos):
    pos_ext = jnp.concatenate([jnp.zeros((S, D), pos.dtype), pos], axis=0)
    out = _pos_embed(x.astype(jnp.int32).reshape(-1), pos_ext.reshape(-1))
    return out.reshape(B, S, D)
